# baseline (speedup 1.0000x reference)
import jax
import jax.numpy as jnp
from jax.experimental import pallas as pl
from jax.experimental.pallas import tpu as pltpu

_NMS_THRESH = 0.5
_LANES = 128
_SUBLANES = 8
_BIG = 1.0e9


_NCHUNKS = 8


def _nms_kernel(n_boxes, x1_ref, y1_ref, x2_ref, y2_ref, keep_ref,
                area_ref, m_ref, sm_ref):
    shape = x1_ref.shape
    rows = shape[0]
    crows = rows // _NCHUNKS
    f32 = jnp.float32

    def pos_iota(nrows, row0):
        return ((jax.lax.broadcasted_iota(jnp.int32, (nrows, _LANES), 0)
                 + row0) * _LANES
                + jax.lax.broadcasted_iota(
                    jnp.int32, (nrows, _LANES), 1)).astype(f32)

    pos0 = pos_iota(rows, 0)
    valid = pos0 < f32(n_boxes)
    area_ref[...] = (x2_ref[...] - x1_ref[...]) * (y2_ref[...] - y1_ref[...])
    m_ref[...] = jnp.where(valid, pos0, _BIG)
    keep_ref[...] = jnp.where(valid, f32(1.0), f32(0.0))

    lane_iota = jax.lax.broadcasted_iota(
        jnp.int32, (1, _LANES), 1).astype(f32)

    def pick(ref, r, onehot):
        return jnp.sum(ref[pl.ds(r, 1), :] * onehot)

    def cond(next_pos):
        return next_pos < _BIG * 0.5

    def body(next_pos):
        rf = jnp.floor(next_pos * (1.0 / _LANES))
        r = rf.astype(jnp.int32)
        cf = next_pos - rf * _LANES
        onehot = jnp.where(lane_iota == cf, f32(1.0), f32(0.0))
        x1i = pick(x1_ref, r, onehot)
        y1i = pick(y1_ref, r, onehot)
        x2i = pick(x2_ref, r, onehot)
        y2i = pick(y2_ref, r, onehot)
        ai = pick(area_ref, r, onehot)

        for ch in range(_NCHUNKS):
            row0 = ch * crows
            sl = (pl.ds(row0, crows), slice(None))
            sm_ref[ch] = f32(_BIG)

            @pl.when(next_pos < f32((row0 + crows) * _LANES))
            def _():
                x1 = x1_ref[sl]
                y1 = y1_ref[sl]
                x2 = x2_ref[sl]
                y2 = y2_ref[sl]
                area = area_ref[sl]
                xx1 = jnp.maximum(x1i, x1)
                yy1 = jnp.maximum(y1i, y1)
                xx2 = jnp.minimum(x2i, x2)
                yy2 = jnp.minimum(y2i, y2)
                w = jnp.maximum(f32(1e-10), xx2 - xx1)
                h = jnp.maximum(f32(1e-10), yy2 - yy1)
                inter = w * h
                iou = inter / (ai + area - inter + f32(1e-14))

                pos = pos_iota(crows, row0)
                sup = (iou > _NMS_THRESH) & (pos > next_pos)
                keep_ref[sl] = jnp.where(sup, f32(0.0), keep_ref[sl])
                m = jnp.where(sup | (pos == next_pos), f32(_BIG), m_ref[sl])
                m_ref[sl] = m
                sm_ref[ch] = jnp.min(m)

        nxt = sm_ref[0]
        for ch in range(1, _NCHUNKS):
            nxt = jnp.minimum(nxt, sm_ref[ch])
        return nxt

    next0 = jnp.min(m_ref[...])
    jax.lax.while_loop(cond, body, next0)


def kernel(boxes, scores):
    n = boxes.shape[0]
    pad_n = ((n + _LANES * _SUBLANES - 1)
             // (_LANES * _SUBLANES)) * (_LANES * _SUBLANES)
    rows = pad_n // _LANES

    xy1 = boxes[:, :2] - boxes[:, 2:] * 0.5
    xy2 = boxes[:, :2] + boxes[:, 2:] * 0.5
    boxes_xyxy = jnp.concatenate([xy1, xy2], axis=-1)

    order = jnp.argsort(-scores)
    b = boxes_xyxy[order]
    planes = [
        jnp.pad(b[:, k], (0, pad_n - n)).reshape(rows, _LANES)
        for k in range(4)
    ]

    keep_sorted = pl.pallas_call(
        lambda *refs: _nms_kernel(n, *refs),
        out_shape=jax.ShapeDtypeStruct((rows, _LANES), jnp.float32),
        scratch_shapes=[
            pltpu.VMEM((rows, _LANES), jnp.float32),
            pltpu.VMEM((rows, _LANES), jnp.float32),
            pltpu.SMEM((_NCHUNKS,), jnp.float32),
        ],
    )(*planes)

    keep_s = keep_sorted.reshape(-1)[:n]
    keep = jnp.zeros((n,), boxes.dtype).at[order].set(keep_s)
    out = jnp.concatenate(
        [boxes_xyxy * keep[:, None], (scores * keep)[:, None]], axis=-1)
    return out


# speedup vs baseline: 2.9424x; 2.9424x over previous
import jax
import jax.numpy as jnp
from jax.experimental import pallas as pl
from jax.experimental.pallas import tpu as pltpu

_NMS_THRESH = 0.5
_LANES = 128
_SUBLANES = 8
_BIG = 1.0e9


def _nms_kernel(n_boxes, x1_ref, y1_ref, x2_ref, y2_ref, keep_ref,
                area_ref, m_ref):
    shape = x1_ref.shape
    rows = shape[0]
    f32 = jnp.float32

    def pos_iota():
        return (jax.lax.broadcasted_iota(jnp.int32, shape, 0) * _LANES
                + jax.lax.broadcasted_iota(jnp.int32, shape, 1)).astype(f32)

    pos0 = pos_iota()
    valid = pos0 < f32(n_boxes)
    area_ref[...] = (x2_ref[...] - x1_ref[...]) * (y2_ref[...] - y1_ref[...])
    m_ref[...] = jnp.where(valid, pos0, _BIG)
    keep_ref[...] = jnp.where(valid, f32(1.0), f32(0.0))

    lane_iota = jax.lax.broadcasted_iota(
        jnp.int32, (1, _LANES), 1).astype(f32)

    def pick(p):
        rf = jnp.floor(p * (1.0 / _LANES))
        r = jnp.minimum(rf.astype(jnp.int32), rows - 1)
        cf = p - rf * _LANES
        onehot = jnp.where(lane_iota == cf, f32(1.0), f32(0.0))
        return (jnp.sum(x1_ref[pl.ds(r, 1), :] * onehot),
                jnp.sum(y1_ref[pl.ds(r, 1), :] * onehot),
                jnp.sum(x2_ref[pl.ds(r, 1), :] * onehot),
                jnp.sum(y2_ref[pl.ds(r, 1), :] * onehot),
                jnp.sum(area_ref[pl.ds(r, 1), :] * onehot))

    def iou_terms(x1i, y1i, x2i, y2i, ai, x1, y1, x2, y2, area):
        xx1 = jnp.maximum(x1i, x1)
        yy1 = jnp.maximum(y1i, y1)
        xx2 = jnp.minimum(x2i, x2)
        yy2 = jnp.minimum(y2i, y2)
        w = jnp.maximum(f32(1e-10), xx2 - xx1)
        h = jnp.maximum(f32(1e-10), yy2 - yy1)
        inter = w * h
        return inter / (ai + area - inter + f32(1e-14))

    def cond(p1):
        return p1 < _BIG * 0.5

    def body(p1):
        m_cur = m_ref[...]
        p2 = jnp.min(jnp.where(m_cur == p1, f32(_BIG), m_cur))
        b1 = pick(p1)
        b2 = pick(p2)

        x1 = x1_ref[...]
        y1 = y1_ref[...]
        x2 = x2_ref[...]
        y2 = y2_ref[...]
        area = area_ref[...]
        pos = pos_iota()

        iou1 = iou_terms(*b1, x1, y1, x2, y2, area)
        iou12 = jnp.max(jnp.where(pos == p2, iou1, f32(0.0)))
        p2_acts = jnp.logical_and(p2 < _BIG * 0.5,
                                  jnp.logical_not(iou12 > _NMS_THRESH))
        iou2 = iou_terms(*b2, x1, y1, x2, y2, area)
        sup = (iou1 > _NMS_THRESH) & (pos > p1)
        sup2 = (iou2 > _NMS_THRESH) & (pos > p2) & p2_acts
        sup = sup | sup2
        keep_ref[...] = jnp.where(sup, f32(0.0), keep_ref[...])
        gone = sup | (pos == p1) | ((pos == p2) & p2_acts)
        m = jnp.where(gone, f32(_BIG), m_cur)
        m_ref[...] = m
        return jnp.min(m)

    next0 = jnp.min(m_ref[...])
    jax.lax.while_loop(cond, body, next0)


def kernel(boxes, scores):
    n = boxes.shape[0]
    pad_n = ((n + _LANES * _SUBLANES - 1)
             // (_LANES * _SUBLANES)) * (_LANES * _SUBLANES)
    rows = pad_n // _LANES

    xy1 = boxes[:, :2] - boxes[:, 2:] * 0.5
    xy2 = boxes[:, :2] + boxes[:, 2:] * 0.5
    boxes_xyxy = jnp.concatenate([xy1, xy2], axis=-1)

    order = jnp.argsort(-scores)
    b = boxes_xyxy[order]
    planes = [
        jnp.pad(b[:, k], (0, pad_n - n)).reshape(rows, _LANES)
        for k in range(4)
    ]

    keep_sorted = pl.pallas_call(
        lambda *refs: _nms_kernel(n, *refs),
        out_shape=jax.ShapeDtypeStruct((rows, _LANES), jnp.float32),
        scratch_shapes=[
            pltpu.VMEM((rows, _LANES), jnp.float32),
            pltpu.VMEM((rows, _LANES), jnp.float32),
        ],
    )(*planes)

    keep_s = keep_sorted.reshape(-1)[:n]
    keep = jnp.zeros((n,), boxes.dtype).at[order].set(keep_s)
    out = jnp.concatenate(
        [boxes_xyxy * keep[:, None], (scores * keep)[:, None]], axis=-1)
    return out


# speedup vs baseline: 3.0410x; 1.0335x over previous
import jax
import jax.numpy as jnp
from jax.experimental import pallas as pl
from jax.experimental.pallas import tpu as pltpu

_NMS_THRESH = 0.5
_LANES = 128
_SUBLANES = 8
_BIG = 1.0e9


_KEPT = 2.0e9
_SUPP = 3.0e9


def _nms_kernel(n_boxes, x1_ref, y1_ref, x2_ref, y2_ref, keep_ref,
                area_ref, m_ref, pos_ref):
    shape = x1_ref.shape
    rows = shape[0]
    f32 = jnp.float32

    pos0 = (jax.lax.broadcasted_iota(jnp.int32, shape, 0) * _LANES
            + jax.lax.broadcasted_iota(jnp.int32, shape, 1)).astype(f32)
    pos_ref[...] = pos0
    valid = pos0 < f32(n_boxes)
    area_ref[...] = (x2_ref[...] - x1_ref[...]) * (y2_ref[...] - y1_ref[...])
    m_ref[...] = jnp.where(valid, pos0, f32(_SUPP))

    lane_iota = jax.lax.broadcasted_iota(
        jnp.int32, (1, _LANES), 1).astype(f32)

    def pick(p):
        rf = jnp.floor(p * (1.0 / _LANES))
        r = jnp.minimum(rf.astype(jnp.int32), rows - 1)
        cf = p - rf * _LANES
        onehot = jnp.where(lane_iota == cf, f32(1.0), f32(0.0))
        return (jnp.sum(x1_ref[pl.ds(r, 1), :] * onehot),
                jnp.sum(y1_ref[pl.ds(r, 1), :] * onehot),
                jnp.sum(x2_ref[pl.ds(r, 1), :] * onehot),
                jnp.sum(y2_ref[pl.ds(r, 1), :] * onehot),
                jnp.sum(area_ref[pl.ds(r, 1), :] * onehot))

    def iou_terms(x1i, y1i, x2i, y2i, ai, x1, y1, x2, y2, area):
        xx1 = jnp.maximum(x1i, x1)
        yy1 = jnp.maximum(y1i, y1)
        xx2 = jnp.minimum(x2i, x2)
        yy2 = jnp.minimum(y2i, y2)
        w = jnp.maximum(f32(1e-10), xx2 - xx1)
        h = jnp.maximum(f32(1e-10), yy2 - yy1)
        inter = w * h
        return inter / (ai + area - inter + f32(1e-14))

    def cond(p1):
        return p1 < _BIG * 0.5

    def body(p1):
        m_cur = m_ref[...]
        p2 = jnp.min(jnp.where(m_cur == p1, f32(_SUPP), m_cur))
        b1 = pick(p1)
        b2 = pick(p2)

        x1 = x1_ref[...]
        y1 = y1_ref[...]
        x2 = x2_ref[...]
        y2 = y2_ref[...]
        area = area_ref[...]
        pos = pos_ref[...]

        iou1 = iou_terms(*b1, x1, y1, x2, y2, area)
        iou12 = jnp.max(jnp.where(pos == p2, iou1, f32(0.0)))
        p2_acts = jnp.logical_and(p2 < _BIG * 0.5,
                                  jnp.logical_not(iou12 > _NMS_THRESH))
        iou2 = iou_terms(*b2, x1, y1, x2, y2, area)
        sup = (iou1 > _NMS_THRESH) & (pos > p1)
        sup2 = (iou2 > _NMS_THRESH) & (pos > p2) & p2_acts
        sup = sup | sup2
        picked = (pos == p1) | ((pos == p2) & p2_acts)
        m = jnp.where(sup, f32(_SUPP),
                      jnp.where(picked, f32(_KEPT), m_cur))
        m_ref[...] = m
        return jnp.min(m)

    next0 = jnp.min(m_ref[...])
    jax.lax.while_loop(cond, body, next0)
    keep_ref[...] = jnp.where(m_ref[...] == f32(_KEPT), f32(1.0), f32(0.0))


def kernel(boxes, scores):
    n = boxes.shape[0]
    pad_n = ((n + _LANES * _SUBLANES - 1)
             // (_LANES * _SUBLANES)) * (_LANES * _SUBLANES)
    rows = pad_n // _LANES

    xy1 = boxes[:, :2] - boxes[:, 2:] * 0.5
    xy2 = boxes[:, :2] + boxes[:, 2:] * 0.5
    boxes_xyxy = jnp.concatenate([xy1, xy2], axis=-1)

    order = jnp.argsort(-scores)
    b = boxes_xyxy[order]
    planes = [
        jnp.pad(b[:, k], (0, pad_n - n)).reshape(rows, _LANES)
        for k in range(4)
    ]

    keep_sorted = pl.pallas_call(
        lambda *refs: _nms_kernel(n, *refs),
        out_shape=jax.ShapeDtypeStruct((rows, _LANES), jnp.float32),
        scratch_shapes=[
            pltpu.VMEM((rows, _LANES), jnp.float32),
            pltpu.VMEM((rows, _LANES), jnp.float32),
            pltpu.VMEM((rows, _LANES), jnp.float32),
        ],
    )(*planes)

    keep_s = keep_sorted.reshape(-1)[:n]
    keep = jnp.zeros((n,), boxes.dtype).at[order].set(keep_s)
    out = jnp.concatenate(
        [boxes_xyxy * keep[:, None], (scores * keep)[:, None]], axis=-1)
    return out
